# Initial kernel scaffold; baseline (speedup 1.0000x reference)
import jax
import jax.numpy as jnp
from jax.experimental import pallas as pl


def kernel(daytime, emb_day, emb_time):
    raise NotImplementedError("write your pallas kernel here")


# speedup vs baseline: 4.4360x; 4.4360x over previous
import jax
import jax.numpy as jnp
from jax import lax
from jax.experimental import pallas as pl
from jax.experimental.pallas import tpu as pltpu
from jax.experimental.pallas import tpu_sc as plsc

B, L = 16384, 200
DAY_VOCAB, TIME_VOCAB = 7, 96
D = 64
N = B * L
NW = 32
POS_PER_W = N // NW
C = 256
G = C // 128
NCHUNKS = POS_PER_W // C


def _sc_body(dt_hbm, cat_hbm, out_hbm, idxraw, idxf, rows, gsem):
    nc = 2
    wid = lax.axis_index("s") * nc + lax.axis_index("c")
    base0 = wid * POS_PER_W
    iota = lax.broadcasted_iota(jnp.int32, (16,), 0)
    even = iota * 2

    def chunk(ci, carry):
        base = base0 + ci * C
        pltpu.sync_copy(dt_hbm.at[pl.ds(2 * base, 2 * C)], idxraw)
        for k in range(C // 16):
            w = k * 32 + even
            d = plsc.load_gather(idxraw, [w])
            t = plsc.load_gather(idxraw, [w + 1])
            j, c = k // 8, (k % 8) * 16
            idxf[j, pl.ds(c, 16)] = d * TIME_VOCAB + t
        copies = []
        for g in range(G):
            copies.append(
                pltpu.async_copy(
                    cat_hbm.at[idxf.at[g]], rows.at[pl.ds(g * 128, 128)], gsem
                )
            )
        for cp in copies:
            cp.wait()
        pltpu.sync_copy(rows, out_hbm.at[pl.ds(base, C)])
        return carry

    lax.fori_loop(0, NCHUNKS, chunk, None)


@jax.jit
def _daytime_sc(dt_flat, cat):
    mesh = plsc.VectorSubcoreMesh(core_axis_name="c", subcore_axis_name="s")
    return pl.kernel(
        _sc_body,
        out_type=jax.ShapeDtypeStruct((N, 2 * D), jnp.float32),
        mesh=mesh,
        compiler_params=pltpu.CompilerParams(needs_layout_passes=False),
        scratch_types=[
            pltpu.VMEM((2 * C,), jnp.int32),
            pltpu.VMEM((G, 128), jnp.int32),
            pltpu.VMEM((C, 2 * D), jnp.float32),
            pltpu.SemaphoreType.DMA,
        ],
    )(dt_flat, cat)


def kernel(daytime, emb_day, emb_time):
    cat = jnp.concatenate(
        (
            jnp.broadcast_to(emb_day[:, None, :], (DAY_VOCAB, TIME_VOCAB, D)),
            jnp.broadcast_to(emb_time[None, :, :], (DAY_VOCAB, TIME_VOCAB, D)),
        ),
        axis=-1,
    ).reshape(DAY_VOCAB * TIME_VOCAB, 2 * D)
    dt_flat = daytime.reshape(2 * N)
    out = _daytime_sc(dt_flat, cat)
    return out.reshape(B, L, 2 * D)


# speedup vs baseline: 4.5242x; 1.0199x over previous
import jax
import jax.numpy as jnp
from jax import lax
from jax.experimental import pallas as pl
from jax.experimental.pallas import tpu as pltpu
from jax.experimental.pallas import tpu_sc as plsc

B, L = 16384, 200
DAY_VOCAB, TIME_VOCAB = 7, 96
D = 64
N = B * L
NW = 32
POS_PER_W = N // NW
C = 256
G = C // 128
NCHUNKS = POS_PER_W // C


def _sc_body(dt_hbm, cat_hbm, out_hbm, *s):
    iraw, idxf, rows = (s[0], s[1]), (s[2], s[3]), (s[4], s[5])
    isem, gsem, wsem = (s[6], s[7]), (s[8], s[9]), (s[10], s[11])
    nc = 2
    wid = lax.axis_index("s") * nc + lax.axis_index("c")
    base0 = wid * POS_PER_W
    iota = lax.broadcasted_iota(jnp.int32, (16,), 0)
    even = iota * 2
    ngroups = NCHUNKS // 2

    def idx_copy(b, ci):
        return pltpu.make_async_copy(
            dt_hbm.at[pl.ds(2 * (base0 + ci * C), 2 * C)], iraw[b], isem[b]
        )

    def write_copy(b, ci):
        return pltpu.make_async_copy(
            rows[b], out_hbm.at[pl.ds(base0 + ci * C, C)], wsem[b]
        )

    idx_copy(0, 0).start()
    idx_copy(1, 1).start()

    def group(g, carry):
        for b in range(2):
            ci = 2 * g + b
            idx_copy(b, ci).wait()
            for k in range(C // 16):
                w = k * 32 + even
                d = plsc.load_gather(iraw[b], [w])
                t = plsc.load_gather(iraw[b], [w + 1])
                j, c = k // 8, (k % 8) * 16
                idxf[b][j, pl.ds(c, 16)] = d * TIME_VOCAB + t

            @pl.when(g < ngroups - 1)
            def _():
                idx_copy(b, ci + 2).start()

            @pl.when(g >= 1)
            def _():
                write_copy(b, ci).wait()

            cps = [
                pltpu.async_copy(
                    cat_hbm.at[idxf[b].at[gg]],
                    rows[b].at[pl.ds(gg * 128, 128)],
                    gsem[b],
                )
                for gg in range(G)
            ]
            for cp in cps:
                cp.wait()
            write_copy(b, ci).start()
        return carry

    lax.fori_loop(0, ngroups, group, None)
    write_copy(0, NCHUNKS - 2).wait()
    write_copy(1, NCHUNKS - 1).wait()


@jax.jit
def _daytime_sc(dt_flat, cat):
    mesh = plsc.VectorSubcoreMesh(core_axis_name="c", subcore_axis_name="s")
    return pl.kernel(
        _sc_body,
        out_type=jax.ShapeDtypeStruct((N, 2 * D), jnp.float32),
        mesh=mesh,
        compiler_params=pltpu.CompilerParams(needs_layout_passes=False),
        scratch_types=(
            [pltpu.VMEM((2 * C,), jnp.int32)] * 2
            + [pltpu.VMEM((G, 128), jnp.int32)] * 2
            + [pltpu.VMEM((C, 2 * D), jnp.float32)] * 2
            + [pltpu.SemaphoreType.DMA] * 6
        ),
    )(dt_flat, cat)


def kernel(daytime, emb_day, emb_time):
    cat = jnp.concatenate(
        (
            jnp.broadcast_to(emb_day[:, None, :], (DAY_VOCAB, TIME_VOCAB, D)),
            jnp.broadcast_to(emb_time[None, :, :], (DAY_VOCAB, TIME_VOCAB, D)),
        ),
        axis=-1,
    ).reshape(DAY_VOCAB * TIME_VOCAB, 2 * D)
    dt_flat = daytime.reshape(2 * N)
    out = _daytime_sc(dt_flat, cat)
    return out.reshape(B, L, 2 * D)


# speedup vs baseline: 5.6755x; 1.2545x over previous
import jax
import jax.numpy as jnp
from jax import lax
from jax.experimental import pallas as pl
from jax.experimental.pallas import tpu as pltpu
from jax.experimental.pallas import tpu_sc as plsc

B, L = 16384, 200
DAY_VOCAB, TIME_VOCAB = 7, 96
D = 64
N = B * L
NW = 32
POS_PER_W = N // NW
C = 256
G = C // 128
NCHUNKS = POS_PER_W // C


def _sc_body(dt_hbm, cat_hbm, out_hbm, *s):
    iraw, idxf, rows = (s[0], s[1]), (s[2], s[3]), (s[4], s[5])
    isem, gsem, wsem = (s[6], s[7]), (s[8], s[9]), (s[10], s[11])
    cat_sp = s[12]
    nc = 2
    wid = lax.axis_index("s") * nc + lax.axis_index("c")
    base0 = wid * POS_PER_W
    iota = lax.broadcasted_iota(jnp.int32, (16,), 0)
    even = iota * 2
    ngroups = NCHUNKS // 2

    @pl.when(lax.axis_index("s") == 0)
    def _():
        pltpu.sync_copy(cat_hbm, cat_sp)

    plsc.subcore_barrier()

    def idx_copy(b, ci):
        return pltpu.make_async_copy(
            dt_hbm.at[pl.ds(2 * (base0 + ci * C), 2 * C)], iraw[b], isem[b]
        )

    def write_copy(b, ci):
        return pltpu.make_async_copy(
            rows[b], out_hbm.at[pl.ds(base0 + ci * C, C)], wsem[b]
        )

    idx_copy(0, 0).start()
    idx_copy(1, 1).start()

    def group(g, carry):
        for b in range(2):
            ci = 2 * g + b
            idx_copy(b, ci).wait()
            for k in range(C // 16):
                w = k * 32 + even
                d = plsc.load_gather(iraw[b], [w])
                t = plsc.load_gather(iraw[b], [w + 1])
                j, c = k // 8, (k % 8) * 16
                idxf[b][j, pl.ds(c, 16)] = d * TIME_VOCAB + t

            @pl.when(g < ngroups - 1)
            def _():
                idx_copy(b, ci + 2).start()

            @pl.when(g >= 1)
            def _():
                write_copy(b, ci).wait()

            cps = [
                pltpu.async_copy(
                    cat_sp.at[idxf[b].at[gg]],
                    rows[b].at[pl.ds(gg * 128, 128)],
                    gsem[b],
                )
                for gg in range(G)
            ]
            for cp in cps:
                cp.wait()
            write_copy(b, ci).start()
        return carry

    lax.fori_loop(0, ngroups, group, None)
    write_copy(0, NCHUNKS - 2).wait()
    write_copy(1, NCHUNKS - 1).wait()


@jax.jit
def _daytime_sc(dt_flat, cat):
    mesh = plsc.VectorSubcoreMesh(core_axis_name="c", subcore_axis_name="s")
    return pl.kernel(
        _sc_body,
        out_type=jax.ShapeDtypeStruct((N, 2 * D), jnp.float32),
        mesh=mesh,
        compiler_params=pltpu.CompilerParams(needs_layout_passes=False),
        scratch_types=(
            [pltpu.VMEM((2 * C,), jnp.int32)] * 2
            + [pltpu.VMEM((G, 128), jnp.int32)] * 2
            + [pltpu.VMEM((C, 2 * D), jnp.float32)] * 2
            + [pltpu.SemaphoreType.DMA] * 6
            + [pltpu.VMEM_SHARED((DAY_VOCAB * TIME_VOCAB, 2 * D), jnp.float32)]
        ),
    )(dt_flat, cat)


def kernel(daytime, emb_day, emb_time):
    cat = jnp.concatenate(
        (
            jnp.broadcast_to(emb_day[:, None, :], (DAY_VOCAB, TIME_VOCAB, D)),
            jnp.broadcast_to(emb_time[None, :, :], (DAY_VOCAB, TIME_VOCAB, D)),
        ),
        axis=-1,
    ).reshape(DAY_VOCAB * TIME_VOCAB, 2 * D)
    dt_flat = daytime.reshape(2 * N)
    out = _daytime_sc(dt_flat, cat)
    return out.reshape(B, L, 2 * D)
